# Initial kernel scaffold; baseline (speedup 1.0000x reference)
import jax
import jax.numpy as jnp
from jax.experimental import pallas as pl


def kernel(x_user, x_item, edge_u2i, edge_i2u, emb_W_user, emb_b_user, emb_W_item, emb_b_item, Wl1_u, bl1_u, Wr1_u, Wl1_i, bl1_i, Wr1_i, Wl2_u, bl2_u, Wr2_u, Wl2_i, bl2_i, Wr2_i, Wm, bm):
    raise NotImplementedError("write your pallas kernel here")


# speedup vs baseline: 13.5436x; 13.5436x over previous
import functools

import jax
import jax.numpy as jnp
from jax import lax
from jax.experimental import pallas as pl
from jax.experimental.pallas import tpu as pltpu
from jax.experimental.pallas import tpu_sc as plsc

_N = 25000
_E = 312500
_NCOL = 4
_DIM = 32
_NPAD = 25088
_NC = 2
_NS = 16
_NW = _NC * _NS
_ROWS_SUB = _NPAD // _NS
_MICRO = 128
_KM = 8
_CH = _MICRO * _KM
_NCHUNK = 10
_EPT = _CH * _NCHUNK
_EPAD = _EPT * _NW
_W = 16


def _sc_segsum(table, src2d, dst2d, zeros):
    mesh = plsc.VectorSubcoreMesh(core_axis_name="c", subcore_axis_name="s")

    @functools.partial(
        pl.kernel,
        mesh=mesh,
        compiler_params=pltpu.CompilerParams(use_tc_tiling_on_sc=False),
        out_type=jax.ShapeDtypeStruct((_NC * _NPAD, _W), jnp.float32),
        scratch_types=[
            pltpu.VMEM((_KM, _MICRO), jnp.int32),
            pltpu.VMEM((_KM, _MICRO), jnp.int32),
            pltpu.VMEM((_CH, _W), jnp.float32),
            pltpu.VMEM_SHARED((_NPAD, _W), jnp.float32),
            pltpu.SemaphoreType.DMA,
        ],
    )
    def k(table_hbm, src_hbm, dst_hbm, zeros_hbm, out_hbm,
          src_v, dst_v, rows_v, acc_sh, sem):
        cid = lax.axis_index("c")
        sid = lax.axis_index("s")
        wid = sid * _NC + cid
        pltpu.sync_copy(zeros_hbm, acc_sh.at[pl.ds(sid * _ROWS_SUB, _ROWS_SUB)])
        plsc.subcore_barrier()

        def body(g, carry):
            r0 = wid * (_EPT // _MICRO) + g * _KM
            pltpu.sync_copy(src_hbm.at[pl.ds(r0, _KM)], src_v)
            pltpu.sync_copy(dst_hbm.at[pl.ds(r0, _KM)], dst_v)
            cps = [
                pltpu.async_copy(
                    table_hbm.at[src_v.at[km]],
                    rows_v.at[pl.ds(km * _MICRO, _MICRO)],
                    sem,
                )
                for km in range(_KM)
            ]
            for cp in cps:
                cp.wait()
            for km in range(_KM):
                pltpu.sync_copy(
                    rows_v.at[pl.ds(km * _MICRO, _MICRO)],
                    acc_sh.at[dst_v.at[km]],
                    add=True,
                )
            return carry

        lax.fori_loop(0, _NCHUNK, body, 0)
        plsc.subcore_barrier()
        pltpu.sync_copy(
            acc_sh.at[pl.ds(sid * _ROWS_SUB, _ROWS_SUB)],
            out_hbm.at[pl.ds(cid * _NPAD + sid * _ROWS_SUB, _ROWS_SUB)],
        )

    return k(table, src2d, dst2d, zeros)


_RBLK = 3136


def _affine_body(p_ref, x_ref, a_ref, b_ref, c_ref, o_ref):
    s = p_ref[0] + p_ref[1]
    hp = jax.lax.Precision.HIGHEST
    o_ref[...] = (
        jnp.dot(s, a_ref[...], preferred_element_type=jnp.float32, precision=hp)
        + jnp.dot(x_ref[...], b_ref[...], preferred_element_type=jnp.float32,
                  precision=hp)
        + c_ref[...]
    )


def _affine(parts, x, A, B, c):
    oc = A.shape[1]
    return pl.pallas_call(
        _affine_body,
        grid=(_NPAD // _RBLK,),
        in_specs=[
            pl.BlockSpec((_NC, _RBLK, _W), lambda i: (0, i, 0)),
            pl.BlockSpec((_RBLK, _NCOL), lambda i: (i, 0)),
            pl.BlockSpec((_W, oc), lambda i: (0, 0)),
            pl.BlockSpec((_NCOL, oc), lambda i: (0, 0)),
            pl.BlockSpec((1, oc), lambda i: (0, 0)),
        ],
        out_specs=pl.BlockSpec((_RBLK, oc), lambda i: (i, 0)),
        out_shape=jax.ShapeDtypeStruct((_NPAD, oc), jnp.float32),
    )(parts, x, A, B, c)


def _fold(M, eW, eb):
    M4 = M.reshape(_NCOL, _DIM, -1)
    A = jnp.einsum("ck,cko->co", eW, M4)
    b = jnp.einsum("ck,cko->o", eb, M4)
    return A, b


def _pad_edges(src, dst):
    npad = _EPAD - _E
    pad_src = jnp.full((npad,), _N, dtype=jnp.int32)
    pad_dst = _N + (jnp.arange(npad, dtype=jnp.int32) % (_NPAD - _N))
    src_p = jnp.concatenate([src.astype(jnp.int32), pad_src])
    dst_p = jnp.concatenate([dst.astype(jnp.int32), pad_dst])
    return src_p.reshape(-1, _MICRO), dst_p.reshape(-1, _MICRO)


def kernel(x_user, x_item, edge_u2i, edge_i2u,
           emb_W_user, emb_b_user, emb_W_item, emb_b_item,
           Wl1_u, bl1_u, Wr1_u, Wl1_i, bl1_i, Wr1_i,
           Wl2_u, bl2_u, Wr2_u, Wl2_i, bl2_i, Wr2_i,
           Wm, bm):
    f32 = jnp.float32

    def bf(w):
        return w.astype(jnp.bfloat16).astype(f32)

    with jax.default_matmul_precision("highest"):
        G = bf(Wl2_u) @ bf(Wm)
        H = bf(Wr2_u) @ bf(Wm)
        c0 = bl2_u @ bf(Wm) + bm
        A1, b1 = _fold(bf(Wl1_i) @ G, emb_W_user, emb_b_user)
        A2, b2 = _fold(bf(Wr1_i) @ G, emb_W_item, emb_b_item)
        cJ = b2 + bl1_i @ G
        A3, b3 = _fold(bf(Wl1_u) @ H, emb_W_item, emb_b_item)
        A4, b4 = _fold(bf(Wr1_u) @ H, emb_W_user, emb_b_user)
        cF = b4 + bl1_u @ H + c0

    OUT = Wm.shape[1]
    A_J = jnp.zeros((_W, _W), f32)
    A_J = A_J.at[0:4, 4:14].set(A1).at[4, 4:14].set(b1)
    B_J = jnp.zeros((_NCOL, _W), f32)
    B_J = B_J.at[:, 0:4].set(jnp.eye(_NCOL, dtype=f32)).at[:, 4:14].set(A2)
    c_J = jnp.zeros((1, _W), f32).at[0, 4:14].set(cJ).at[0, 14].set(1.0)
    A_F = jnp.zeros((_W, OUT), f32)
    A_F = A_F.at[0:4, :].set(A3).at[4:14, :].set(jnp.eye(10, dtype=f32))
    A_F = A_F.at[14, :].set(b3)
    B_F = A4.astype(f32)
    c_F = cF.reshape(1, OUT).astype(f32)

    zeros = jnp.zeros((_ROWS_SUB, _W), f32)
    xu_pad = jnp.zeros((_NPAD, _NCOL), f32).at[:_N].set(x_user)
    xi_pad = jnp.zeros((_NPAD, _NCOL), f32).at[:_N].set(x_item)
    packed_u = jnp.zeros((_NPAD, _W), f32)
    packed_u = packed_u.at[:_N, 0:4].set(x_user).at[:_N, 4].set(1.0)
    srcA, dstA = _pad_edges(edge_u2i[0], edge_u2i[1])
    srcB, dstB = _pad_edges(edge_i2u[0], edge_i2u[1])

    partsA = _sc_segsum(packed_u, srcA, dstA, zeros).reshape(_NC, _NPAD, _W)
    j16 = _affine(partsA, xi_pad, A_J, B_J, c_J)
    partsB = _sc_segsum(j16, srcB, dstB, zeros).reshape(_NC, _NPAD, _W)
    outp = _affine(partsB, xu_pad, A_F, B_F, c_F)
    return outp[:_N]


# speedup vs baseline: 13.7345x; 1.0141x over previous
import functools

import jax
import jax.numpy as jnp
from jax import lax
from jax.experimental import pallas as pl
from jax.experimental.pallas import tpu as pltpu
from jax.experimental.pallas import tpu_sc as plsc

_N = 25000
_E = 312500
_NCOL = 4
_DIM = 32
_NPAD = 25088
_NC = 2
_NS = 16
_NW = _NC * _NS
_ROWS_SUB = _NPAD // _NS
_MICRO = 128
_KM = 8
_CH = _MICRO * _KM
_NCHUNK = 10
_EPT = _CH * _NCHUNK
_EPAD = _EPT * _NW
_W = 16


def _sc_segsum(table, src2d, dst2d, zeros):
    mesh = plsc.VectorSubcoreMesh(core_axis_name="c", subcore_axis_name="s")

    @functools.partial(
        pl.kernel,
        mesh=mesh,
        compiler_params=pltpu.CompilerParams(use_tc_tiling_on_sc=False),
        out_type=jax.ShapeDtypeStruct((_NC * _NPAD, _W), jnp.float32),
        scratch_types=[
            pltpu.VMEM((_EPT,), jnp.int32),
            pltpu.VMEM((_EPT // _MICRO, _MICRO), jnp.int32),
            pltpu.VMEM((_CH, _W), jnp.float32),
            pltpu.VMEM((_CH, _W), jnp.float32),
            pltpu.VMEM_SHARED((_NPAD, _W), jnp.float32),
            pltpu.SemaphoreType.DMA,
            pltpu.SemaphoreType.DMA,
            pltpu.SemaphoreType.DMA,
            pltpu.SemaphoreType.DMA,
        ],
    )
    def k(table_hbm, src_hbm, dst_hbm, zeros_hbm, out_hbm,
          src_all, dst_all, rows0, rows1, acc_sh, semg0, semg1, sems0, sems1):
        cid = lax.axis_index("c")
        sid = lax.axis_index("s")
        wid = sid * _NC + cid
        pltpu.sync_copy(zeros_hbm, acc_sh.at[pl.ds(sid * _ROWS_SUB, _ROWS_SUB)])
        pltpu.sync_copy(src_hbm.at[pl.ds(wid * _EPT, _EPT)], src_all)
        pltpu.sync_copy(
            dst_hbm.at[pl.ds(wid * (_EPT // _MICRO), _EPT // _MICRO)], dst_all)
        plsc.subcore_barrier()

        def body(i, carry):
            g0 = 2 * i
            g1 = g0 + 1
            cp0 = pltpu.async_copy(
                table_hbm.at[src_all.at[pl.ds(g0 * _CH, _CH)]], rows0, semg0)
            cp1 = pltpu.async_copy(
                table_hbm.at[src_all.at[pl.ds(g1 * _CH, _CH)]], rows1, semg1)
            cp0.wait()
            for km in range(_KM):
                pltpu.async_copy(
                    rows0.at[pl.ds(km * _MICRO, _MICRO)],
                    acc_sh.at[dst_all.at[g0 * _KM + km]],
                    sems0, add=True)
            cp1.wait()
            for km in range(_KM):
                pltpu.async_copy(
                    rows1.at[pl.ds(km * _MICRO, _MICRO)],
                    acc_sh.at[dst_all.at[g1 * _KM + km]],
                    sems1, add=True)
            pltpu.make_async_copy(table_hbm.at[pl.ds(0, _CH)], rows0, sems0).wait()
            pltpu.make_async_copy(table_hbm.at[pl.ds(0, _CH)], rows1, sems1).wait()
            return carry

        lax.fori_loop(0, _NCHUNK // 2, body, 0)
        plsc.subcore_barrier()
        pltpu.sync_copy(
            acc_sh.at[pl.ds(sid * _ROWS_SUB, _ROWS_SUB)],
            out_hbm.at[pl.ds(cid * _NPAD + sid * _ROWS_SUB, _ROWS_SUB)],
        )

    return k(table, src2d, dst2d, zeros)


_RBLK = 3136


def _affine_body(p_ref, x_ref, a_ref, b_ref, c_ref, o_ref):
    s = p_ref[0] + p_ref[1]
    hp = jax.lax.Precision.HIGHEST
    o_ref[...] = (
        jnp.dot(s, a_ref[...], preferred_element_type=jnp.float32, precision=hp)
        + jnp.dot(x_ref[...], b_ref[...], preferred_element_type=jnp.float32,
                  precision=hp)
        + c_ref[...]
    )


def _affine(parts, x, A, B, c):
    oc = A.shape[1]
    return pl.pallas_call(
        _affine_body,
        grid=(_NPAD // _RBLK,),
        in_specs=[
            pl.BlockSpec((_NC, _RBLK, _W), lambda i: (0, i, 0)),
            pl.BlockSpec((_RBLK, _NCOL), lambda i: (i, 0)),
            pl.BlockSpec((_W, oc), lambda i: (0, 0)),
            pl.BlockSpec((_NCOL, oc), lambda i: (0, 0)),
            pl.BlockSpec((1, oc), lambda i: (0, 0)),
        ],
        out_specs=pl.BlockSpec((_RBLK, oc), lambda i: (i, 0)),
        out_shape=jax.ShapeDtypeStruct((_NPAD, oc), jnp.float32),
    )(parts, x, A, B, c)


def _fold(M, eW, eb):
    M4 = M.reshape(_NCOL, _DIM, -1)
    A = jnp.einsum("ck,cko->co", eW, M4)
    b = jnp.einsum("ck,cko->o", eb, M4)
    return A, b


def _pad_edges(src, dst):
    npad = _EPAD - _E
    pad_src = jnp.full((npad,), _N, dtype=jnp.int32)
    pad_dst = _N + (jnp.arange(npad, dtype=jnp.int32) % (_NPAD - _N))
    src_p = jnp.concatenate([src.astype(jnp.int32), pad_src])
    dst_p = jnp.concatenate([dst.astype(jnp.int32), pad_dst])
    return src_p, dst_p.reshape(-1, _MICRO)


def kernel(x_user, x_item, edge_u2i, edge_i2u,
           emb_W_user, emb_b_user, emb_W_item, emb_b_item,
           Wl1_u, bl1_u, Wr1_u, Wl1_i, bl1_i, Wr1_i,
           Wl2_u, bl2_u, Wr2_u, Wl2_i, bl2_i, Wr2_i,
           Wm, bm):
    f32 = jnp.float32

    def bf(w):
        return w.astype(jnp.bfloat16).astype(f32)

    with jax.default_matmul_precision("highest"):
        G = bf(Wl2_u) @ bf(Wm)
        H = bf(Wr2_u) @ bf(Wm)
        c0 = bl2_u @ bf(Wm) + bm
        A1, b1 = _fold(bf(Wl1_i) @ G, emb_W_user, emb_b_user)
        A2, b2 = _fold(bf(Wr1_i) @ G, emb_W_item, emb_b_item)
        cJ = b2 + bl1_i @ G
        A3, b3 = _fold(bf(Wl1_u) @ H, emb_W_item, emb_b_item)
        A4, b4 = _fold(bf(Wr1_u) @ H, emb_W_user, emb_b_user)
        cF = b4 + bl1_u @ H + c0

    OUT = Wm.shape[1]
    A_J = jnp.zeros((_W, _W), f32)
    A_J = A_J.at[0:4, 4:14].set(A1).at[4, 4:14].set(b1)
    B_J = jnp.zeros((_NCOL, _W), f32)
    B_J = B_J.at[:, 0:4].set(jnp.eye(_NCOL, dtype=f32)).at[:, 4:14].set(A2)
    c_J = jnp.zeros((1, _W), f32).at[0, 4:14].set(cJ).at[0, 14].set(1.0)
    A_F = jnp.zeros((_W, OUT), f32)
    A_F = A_F.at[0:4, :].set(A3).at[4:14, :].set(jnp.eye(10, dtype=f32))
    A_F = A_F.at[14, :].set(b3)
    B_F = A4.astype(f32)
    c_F = cF.reshape(1, OUT).astype(f32)

    zeros = jnp.zeros((_ROWS_SUB, _W), f32)
    xu_pad = jnp.zeros((_NPAD, _NCOL), f32).at[:_N].set(x_user)
    xi_pad = jnp.zeros((_NPAD, _NCOL), f32).at[:_N].set(x_item)
    packed_u = jnp.zeros((_NPAD, _W), f32)
    packed_u = packed_u.at[:_N, 0:4].set(x_user).at[:_N, 4].set(1.0)
    srcA, dstA = _pad_edges(edge_u2i[0], edge_u2i[1])
    srcB, dstB = _pad_edges(edge_i2u[0], edge_i2u[1])

    partsA = _sc_segsum(packed_u, srcA, dstA, zeros).reshape(_NC, _NPAD, _W)
    j16 = _affine(partsA, xi_pad, A_J, B_J, c_J)
    partsB = _sc_segsum(j16, srcB, dstB, zeros).reshape(_NC, _NPAD, _W)
    outp = _affine(partsB, xu_pad, A_F, B_F, c_F)
    return outp[:_N]


# speedup vs baseline: 18.6523x; 1.3581x over previous
import functools

import jax
import jax.numpy as jnp
from jax import lax
from jax.experimental import pallas as pl
from jax.experimental.pallas import tpu as pltpu
from jax.experimental.pallas import tpu_sc as plsc

_N = 25000
_E = 312500
_NCOL = 4
_DIM = 32
_NPAD = 25088
_NR8 = _NPAD // 8
_NC = 2
_NS = 16
_ROWS_SUB = _NPAD // _NS
_MICRO = 128
_KM = 8
_CH = _MICRO * _KM
_CHUNKS = 320
_EPAD = _CHUNKS * _CH
_NB0 = 12
_NB1 = 8
_W = 16


def _sc_segsum(table, src, dst2d, zeros):
    mesh = plsc.VectorSubcoreMesh(core_axis_name="c", subcore_axis_name="s")

    @functools.partial(
        pl.kernel,
        mesh=mesh,
        compiler_params=pltpu.CompilerParams(use_tc_tiling_on_sc=False),
        out_type=jax.ShapeDtypeStruct((_NC * _NPAD, _W), jnp.float32),
        scratch_types=[
            pltpu.VMEM((_NB0 * _CH,), jnp.int32),
            pltpu.VMEM((_NB0 * _KM, _MICRO), jnp.int32),
            pltpu.VMEM((_CH, _W), jnp.float32),
            pltpu.VMEM((_CH, _W), jnp.float32),
            pltpu.VMEM_SHARED((_NPAD, _W), jnp.float32),
            pltpu.SemaphoreType.DMA,
            pltpu.SemaphoreType.DMA,
            pltpu.SemaphoreType.DMA,
            pltpu.SemaphoreType.DMA,
        ],
    )
    def k(table_hbm, src_hbm, dst_hbm, zeros_hbm, out_hbm,
          src_all, dst_all, rows0, rows1, acc_sh, semg0, semg1, sems0, sems1):
        cid = lax.axis_index("c")
        sid = lax.axis_index("s")
        pltpu.sync_copy(zeros_hbm, acc_sh.at[pl.ds(sid * _ROWS_SUB, _ROWS_SUB)])
        plsc.subcore_barrier()

        def run(base_chunk, nb):
            pltpu.sync_copy(src_hbm.at[pl.ds(base_chunk * _CH, nb * _CH)],
                            src_all.at[pl.ds(0, nb * _CH)])
            pltpu.sync_copy(dst_hbm.at[pl.ds(base_chunk * _KM, nb * _KM)],
                            dst_all.at[pl.ds(0, nb * _KM)])

            def body(i, carry):
                g0 = 2 * i
                g1 = g0 + 1
                cp0 = pltpu.async_copy(
                    table_hbm.at[src_all.at[pl.ds(g0 * _CH, _CH)]], rows0, semg0)
                cp1 = pltpu.async_copy(
                    table_hbm.at[src_all.at[pl.ds(g1 * _CH, _CH)]], rows1, semg1)
                cp0.wait()
                for km in range(_KM):
                    pltpu.async_copy(
                        rows0.at[pl.ds(km * _MICRO, _MICRO)],
                        acc_sh.at[dst_all.at[g0 * _KM + km]],
                        sems0, add=True)
                cp1.wait()
                for km in range(_KM):
                    pltpu.async_copy(
                        rows1.at[pl.ds(km * _MICRO, _MICRO)],
                        acc_sh.at[dst_all.at[g1 * _KM + km]],
                        sems1, add=True)
                pltpu.make_async_copy(
                    table_hbm.at[pl.ds(0, _CH)], rows0, sems0).wait()
                pltpu.make_async_copy(
                    table_hbm.at[pl.ds(0, _CH)], rows1, sems1).wait()
                return carry

            lax.fori_loop(0, nb // 2, body, 0)

        @pl.when(cid == 0)
        def _():
            run(sid * _NB0, _NB0)

        @pl.when(cid == 1)
        def _():
            run(_NS * _NB0 + sid * _NB1, _NB1)

        plsc.subcore_barrier()
        pltpu.sync_copy(
            acc_sh.at[pl.ds(sid * _ROWS_SUB, _ROWS_SUB)],
            out_hbm.at[pl.ds(cid * _NPAD + sid * _ROWS_SUB, _ROWS_SUB)],
        )

    return k(table, src, dst2d, zeros)


def _affine_body(p_ref, x_ref, a_ref, b_ref, c_ref, o_ref):
    s = p_ref[0] + p_ref[1]
    hp = jax.lax.Precision.HIGHEST
    o_ref[...] = (
        jnp.dot(s, a_ref[...], preferred_element_type=jnp.float32, precision=hp)
        + jnp.dot(x_ref[...], b_ref[...], preferred_element_type=jnp.float32,
                  precision=hp)
        + c_ref[...]
    )


def _affine(parts, x8, A_big, B_big, c_big):
    return pl.pallas_call(
        _affine_body,
        grid=(1,),
        in_specs=[
            pl.BlockSpec((_NC, _NR8, 128), lambda i: (0, 0, 0)),
            pl.BlockSpec((_NR8, 32), lambda i: (0, 0)),
            pl.BlockSpec((128, 128), lambda i: (0, 0)),
            pl.BlockSpec((32, 128), lambda i: (0, 0)),
            pl.BlockSpec((1, 128), lambda i: (0, 0)),
        ],
        out_specs=pl.BlockSpec((_NR8, 128), lambda i: (0, 0)),
        out_shape=jax.ShapeDtypeStruct((_NR8, 128), jnp.float32),
    )(parts, x8, A_big, B_big, c_big)


def _fold(M, eW, eb):
    M4 = M.reshape(_NCOL, _DIM, -1)
    A = jnp.einsum("ck,cko->co", eW, M4)
    b = jnp.einsum("ck,cko->o", eb, M4)
    return A, b


def _pad_edges(src, dst):
    npad = _EPAD - _E
    pad_src = jnp.full((npad,), _N, dtype=jnp.int32)
    pad_dst = _N + (jnp.arange(npad, dtype=jnp.int32) % (_NPAD - _N))
    src_p = jnp.concatenate([src.astype(jnp.int32), pad_src])
    dst_p = jnp.concatenate([dst.astype(jnp.int32), pad_dst])
    return src_p, dst_p.reshape(-1, _MICRO)


def _x8(x):
    flat = jnp.pad(x.reshape(-1), (0, (_NPAD - _N) * _NCOL))
    return flat.reshape(_NR8, 8 * _NCOL)


_COLMAP = [(j // _NCOL) * _W + (j % _NCOL) for j in range(8 * _NCOL)]
_ONECOLS = [k * _W + _NCOL for k in range(8)]


def kernel(x_user, x_item, edge_u2i, edge_i2u,
           emb_W_user, emb_b_user, emb_W_item, emb_b_item,
           Wl1_u, bl1_u, Wr1_u, Wl1_i, bl1_i, Wr1_i,
           Wl2_u, bl2_u, Wr2_u, Wl2_i, bl2_i, Wr2_i,
           Wm, bm):
    f32 = jnp.float32

    def bf(w):
        return w.astype(jnp.bfloat16).astype(f32)

    with jax.default_matmul_precision("highest"):
        G = bf(Wl2_u) @ bf(Wm)
        H = bf(Wr2_u) @ bf(Wm)
        c0 = bl2_u @ bf(Wm) + bm
        A1, b1 = _fold(bf(Wl1_i) @ G, emb_W_user, emb_b_user)
        A2, b2 = _fold(bf(Wr1_i) @ G, emb_W_item, emb_b_item)
        cJ = b2 + bl1_i @ G
        A3, b3 = _fold(bf(Wl1_u) @ H, emb_W_item, emb_b_item)
        A4, b4 = _fold(bf(Wr1_u) @ H, emb_W_user, emb_b_user)
        cF = b4 + bl1_u @ H + c0

    A_J = jnp.zeros((_W, _W), f32)
    A_J = A_J.at[0:4, 4:14].set(A1).at[4, 4:14].set(b1)
    B_J = jnp.zeros((_NCOL, _W), f32)
    B_J = B_J.at[:, 0:4].set(jnp.eye(_NCOL, dtype=f32)).at[:, 4:14].set(A2)
    c_J = jnp.zeros((1, _W), f32).at[0, 4:14].set(cJ).at[0, 14].set(1.0)
    A_F = jnp.zeros((_W, _W), f32)
    A_F = A_F.at[0:4, 0:10].set(A3).at[4:14, 0:10].set(jnp.eye(10, dtype=f32))
    A_F = A_F.at[14, 0:10].set(b3)
    B_F = jnp.zeros((_NCOL, _W), f32).at[:, 0:10].set(A4)
    c_F = jnp.zeros((1, _W), f32).at[0, 0:10].set(cF)

    eye8 = jnp.eye(8, dtype=f32)
    A_Jb = jnp.kron(eye8, A_J)
    B_Jb = jnp.kron(eye8, B_J)
    c_Jb = jnp.tile(c_J, (1, 8))
    A_Fb = jnp.kron(eye8, A_F)
    B_Fb = jnp.kron(eye8, B_F)
    c_Fb = jnp.tile(c_F, (1, 8))

    zeros = jnp.zeros((_ROWS_SUB, _W), f32)
    xu8 = _x8(x_user)
    xi8 = _x8(x_item)
    pu8 = jnp.zeros((_NR8, 128), f32)
    pu8 = pu8.at[:, jnp.array(_COLMAP)].set(xu8)
    pu8 = pu8.at[:, jnp.array(_ONECOLS)].set(1.0)
    packed_u = pu8.reshape(_NPAD, _W)
    srcA, dstA = _pad_edges(edge_u2i[0], edge_u2i[1])
    srcB, dstB = _pad_edges(edge_i2u[0], edge_i2u[1])

    partsA = _sc_segsum(packed_u, srcA, dstA, zeros).reshape(_NC, _NR8, 128)
    j16 = _affine(partsA, xi8, A_Jb, B_Jb, c_Jb).reshape(_NPAD, _W)
    partsB = _sc_segsum(j16, srcB, dstB, zeros).reshape(_NC, _NR8, 128)
    out8 = _affine(partsB, xu8, A_Fb, B_Fb, c_Fb)
    return out8.reshape(_NPAD, _W)[:_N, :10]
